# baseline (speedup 1.0000x reference)
import functools

import jax
import jax.numpy as jnp
from jax import lax
from jax.experimental import pallas as pl
from jax.experimental.pallas import tpu as pltpu
from jax.experimental.pallas import tpu_sc as plsc

V_MIN = -10.0
V_MAX = 10.0
ATOMS = 51
DZ = (V_MAX - V_MIN) / (ATOMS - 1)
N = 1048576
W = 128

NC = 2
NS = 16
NW = NC * NS
ROWS_W = N // NW
C = 128
NCHUNK = ROWS_W // C
G = C // 16
NBUF = 4

_mesh = plsc.VectorSubcoreMesh(
    core_axis_name="c", subcore_axis_name="s", num_cores=NC, num_subcores=NS
)


@functools.partial(
    pl.kernel,
    out_type=jax.ShapeDtypeStruct((N, W), jnp.float32),
    mesh=_mesh,
    scratch_types=[
        pltpu.VMEM((ROWS_W,), jnp.float32),
    ]
    + [pltpu.VMEM((C, W), jnp.float32) for _ in range(NBUF)]
    + [pltpu.SemaphoreType.DMA for _ in range(NBUF)],
    compiler_params=pltpu.CompilerParams(needs_layout_passes=False),
)
def _c51_sc(s_hbm, out_hbm, s_v, *bufs_and_sems):
    o_bufs = bufs_and_sems[:NBUF]
    sos = bufs_and_sems[NBUF:]
    wid = lax.axis_index("s") * NC + lax.axis_index("c")
    base = wid * ROWS_W
    lanes = lax.iota(jnp.int32, 16)
    zeros16 = jnp.zeros((16,), jnp.float32)

    def o_dst(cc):
        off = pl.multiple_of(base + cc * C, 8)
        return out_hbm.at[pl.ds(off, C)]

    for o_v in o_bufs:
        @pl.loop(0, C)
        def _zrow(r, o_v=o_v):
            for c0 in range(48, W, 16):
                o_v[r, pl.ds(c0, 16)] = zeros16

    pltpu.sync_copy(s_hbm.at[pl.ds(base, ROWS_W)], s_v)

    @pl.loop(0, NCHUNK, step=NBUF)
    def _chunk(c):
        for b in range(NBUF):
            cc = c + b
            o_v, so = o_bufs[b], sos[b]

            @pl.when(cc >= NBUF)
            def _():
                pltpu.make_async_copy(o_v, o_dst(cc), so).wait()

            @pl.loop(0, G)
            def _group(g):
                sv = s_v[pl.ds(cc * C + g * 16, 16)]
                t = jnp.minimum(jnp.maximum(sv, V_MIN), V_MAX)
                bv = (t - V_MIN) / jnp.float32(DZ)
                idx0 = (lanes + g * 16) * W
                for j in range(ATOMS):
                    v = jnp.maximum(1.0 - jnp.abs(bv - jnp.float32(j)), 0.0)
                    idx = idx0 + j
                    plsc.store_scatter(
                        o_v,
                        [
                            lax.shift_right_logical(idx, 7),
                            lax.bitwise_and(idx, 127),
                        ],
                        v,
                    )

            pltpu.async_copy(o_v, o_dst(cc), so)

    for b in range(NBUF):
        pltpu.make_async_copy(
            o_bufs[b], o_dst(NCHUNK - NBUF + b), sos[b]
        ).wait()


def kernel(scalar):
    return _c51_sc(scalar)[:, :ATOMS]


# speedup vs baseline: 2.5076x; 2.5076x over previous
import functools

import jax
import jax.numpy as jnp
from jax import lax
from jax.experimental import pallas as pl
from jax.experimental.pallas import tpu as pltpu
from jax.experimental.pallas import tpu_sc as plsc

V_MIN = -10.0
V_MAX = 10.0
ATOMS = 51
DZ = (V_MAX - V_MIN) / (ATOMS - 1)
N = 1048576
W = 128

NC = 2
NS = 16
NW = NC * NS
ROWS_W = N // NW
C = 128
NCHUNK = ROWS_W // C
G = C // 16
NBUF = 4

_mesh = plsc.VectorSubcoreMesh(
    core_axis_name="c", subcore_axis_name="s", num_cores=NC, num_subcores=NS
)


@functools.partial(
    pl.kernel,
    out_type=jax.ShapeDtypeStruct((N, W), jnp.float32),
    mesh=_mesh,
    scratch_types=[
        pltpu.VMEM((ROWS_W,), jnp.float32),
    ]
    + [pltpu.VMEM((C, W), jnp.float32) for _ in range(NBUF)]
    + [pltpu.VMEM((2, C), jnp.int32) for _ in range(NBUF)]
    + [pltpu.SemaphoreType.DMA for _ in range(NBUF)],
    compiler_params=pltpu.CompilerParams(needs_layout_passes=False),
)
def _c51_sc(s_hbm, out_hbm, s_v, *rest):
    o_bufs = rest[:NBUF]
    i_bufs = rest[NBUF:2 * NBUF]
    sos = rest[2 * NBUF:]
    wid = lax.axis_index("s") * NC + lax.axis_index("c")
    base = wid * ROWS_W
    lanes = lax.iota(jnp.int32, 16)
    lanes128 = lanes * W
    zeros16 = jnp.zeros((16,), jnp.float32)

    def o_dst(cc):
        off = pl.multiple_of(base + cc * C, 8)
        return out_hbm.at[pl.ds(off, C)]

    def scatter2d(o_v, idx, val):
        plsc.store_scatter(
            o_v,
            [lax.shift_right_logical(idx, 7), lax.bitwise_and(idx, 127)],
            val,
        )

    for o_v in o_bufs:
        @pl.loop(0, C)
        def _zrow(r, o_v=o_v):
            for c0 in range(0, W, 16):
                o_v[r, pl.ds(c0, 16)] = zeros16

    pltpu.sync_copy(s_hbm.at[pl.ds(base, ROWS_W)], s_v)

    @pl.loop(0, NCHUNK, step=NBUF)
    def _chunk(c):
        for b in range(NBUF):
            cc = c + b
            o_v, i_v, so = o_bufs[b], i_bufs[b], sos[b]

            @pl.when(cc >= NBUF)
            def _():
                pltpu.make_async_copy(o_v, o_dst(cc), so).wait()

                @pl.loop(0, G)
                def _clean(g):
                    scatter2d(o_v, i_v[0, pl.ds(g * 16, 16)], zeros16)
                    scatter2d(o_v, i_v[1, pl.ds(g * 16, 16)], zeros16)

            @pl.loop(0, G)
            def _group(g):
                sv = s_v[pl.ds(cc * C + g * 16, 16)]
                t = jnp.minimum(jnp.maximum(sv, V_MIN), V_MAX)
                bv = (t - V_MIN) / jnp.float32(DZ)
                lv = bv.astype(jnp.int32)
                frac = bv - lv.astype(jnp.float32)
                lower = 1.0 - frac
                upper = jnp.where(bv < jnp.float32(ATOMS - 1), frac, 0.0)
                idx_l = lanes128 + g * (16 * W) + lv
                idx_u = idx_l + 1
                scatter2d(o_v, idx_u, upper)
                scatter2d(o_v, idx_l, lower)
                i_v[0, pl.ds(g * 16, 16)] = idx_l
                i_v[1, pl.ds(g * 16, 16)] = idx_u

            pltpu.async_copy(o_v, o_dst(cc), so)

    for b in range(NBUF):
        pltpu.make_async_copy(
            o_bufs[b], o_dst(NCHUNK - NBUF + b), sos[b]
        ).wait()


def kernel(scalar):
    return _c51_sc(scalar)[:, :ATOMS]
